# Initial kernel scaffold; baseline (speedup 1.0000x reference)
import jax
import jax.numpy as jnp
from jax.experimental import pallas as pl


def kernel(category_inputs, tables):
    raise NotImplementedError("write your pallas kernel here")


# speedup vs baseline: 1.1483x; 1.1483x over previous
import functools
import math

import jax
import jax.numpy as jnp
from jax import lax
from jax.experimental import pallas as pl
from jax.experimental.pallas import tpu as pltpu
from jax.experimental.pallas import tpu_sc as plsc

_NC = 2
_NS = 16
_NW = _NC * _NS
_L = 16


def _make_gather(N, D, rpw, chunk, period):
    n_chunks = rpw // chunk
    n_groups = rpw // period
    mesh = plsc.VectorSubcoreMesh(core_axis_name="c", subcore_axis_name="s")

    @functools.partial(
        pl.kernel,
        mesh=mesh,
        compiler_params=pltpu.CompilerParams(use_tc_tiling_on_sc=False),
        out_type=jax.ShapeDtypeStruct((N, D), jnp.float32),
        scratch_types=[
            pltpu.VMEM((rpw,), jnp.int32),
            pltpu.VMEM((period,), jnp.int32),
            pltpu.VMEM((chunk, D), jnp.float32),
            pltpu.SemaphoreType.DMA,
        ],
    )
    def gather_kernel(idx_hbm, offs_hbm, table_hbm, out_hbm,
                      idx_v, offs_v, rows_v, sem):
        wid = lax.axis_index("s") * _NC + lax.axis_index("c")
        base = wid * rpw
        pltpu.sync_copy(idx_hbm.at[pl.ds(base, rpw)], idx_v)
        pltpu.sync_copy(offs_hbm, offs_v)

        def add_offsets(g, carry):
            for j in range(period // _L):
                s = pl.ds(g * period + j * _L, _L)
                idx_v[s] = idx_v[s] + offs_v[pl.ds(j * _L, _L)]
            return carry

        lax.fori_loop(0, n_groups, add_offsets, 0)

        for c in range(n_chunks):
            pltpu.async_copy(
                table_hbm.at[idx_v.at[pl.ds(c * chunk, chunk)]],
                rows_v, sem).wait()
            pltpu.sync_copy(rows_v, out_hbm.at[pl.ds(base + c * chunk, chunk)])

    return gather_kernel


def kernel(category_inputs, tables):
    B, F = category_inputs.shape
    _, V, D = tables.shape
    N = B * F
    rpw = N // _NW
    period = math.lcm(_L, F)
    chunk = rpw // 8

    idx_flat = category_inputs.reshape(N)
    table_flat = tables.reshape(F * V, D)
    offs = jnp.tile(jnp.arange(F, dtype=jnp.int32) * V, period // F)

    out = _make_gather(N, D, rpw, chunk, period)(idx_flat, offs, table_flat)
    return out.reshape(B, F * D)


# speedup vs baseline: 6.5150x; 5.6735x over previous
import functools

import jax
import jax.numpy as jnp
from jax import lax
from jax.experimental import pallas as pl
from jax.experimental.pallas import tpu as pltpu
from jax.experimental.pallas import tpu_sc as plsc

_NC = 2
_NS = 16
_NW = _NC * _NS
_L = 16


def _make_rowgather(R, V, B, rows_per_w, chunk):
    n_chunks = B // chunk
    mesh = plsc.VectorSubcoreMesh(core_axis_name="c", subcore_axis_name="s")

    @functools.partial(
        pl.kernel,
        mesh=mesh,
        compiler_params=pltpu.CompilerParams(
            use_tc_tiling_on_sc=True, needs_layout_passes=False),
        out_type=jax.ShapeDtypeStruct((R, B), jnp.float32),
        scratch_types=[
            pltpu.VMEM((V,), jnp.float32),
            pltpu.VMEM((chunk,), jnp.int32),
            pltpu.VMEM((chunk,), jnp.float32),
        ],
    )
    def rowgather_kernel(table_hbm, idx_hbm, out_hbm, row_v, idx_v, res_v):
        wid = lax.axis_index("s") * _NC + lax.axis_index("c")
        for i in range(rows_per_w):
            j = wid * rows_per_w + i
            f = lax.shift_right_logical(j, 4)
            pltpu.sync_copy(table_hbm.at[j], row_v)
            for c in range(n_chunks):
                pltpu.sync_copy(idx_hbm.at[f, pl.ds(c * chunk, chunk)], idx_v)

                def gather_groups(k, carry):
                    for u in range(4):
                        s = pl.ds((k * 4 + u) * _L, _L)
                        res_v[s] = plsc.load_gather(row_v, [idx_v[s]])
                    return carry

                lax.fori_loop(0, chunk // (4 * _L), gather_groups, 0)
                pltpu.sync_copy(res_v, out_hbm.at[j, pl.ds(c * chunk, chunk)])

    return rowgather_kernel


def kernel(category_inputs, tables):
    B, F = category_inputs.shape
    _, V, D = tables.shape
    R = F * D

    table_rows = tables.transpose(0, 2, 1).reshape(R, V)
    idx_t = category_inputs.T

    out_t = _make_rowgather(R, V, B, R // _NW, 4096)(table_rows, idx_t)
    return out_t.T.reshape(B, R)
